# baseline (speedup 1.0000x reference)
import functools

import jax
import jax.numpy as jnp
from jax import lax
from jax.experimental import pallas as pl
from jax.experimental.pallas import tpu as pltpu
from jax.experimental.pallas import tpu_sc as plsc

B = 16384
D = 32
L = 16
BLK = 128
RPB = BLK // D
CH = 128


def _lanesum(q):
    dn = lax.GatherDimensionNumbers(
        offset_dims=(), collapsed_slice_dims=(0,), start_index_map=(0,))
    for s in (1, 2, 4, 8):
        idx = lax.iota(jnp.int32, L) ^ s
        q = q + lax.gather(q, idx[:, None], dn, (1,),
                           mode=lax.GatherScatterMode.PROMISE_IN_BOUNDS)
    return q


def _build(nc: int, ns: int, n_user: int, n_item: int):
    nw = nc * ns
    bpw = B // nw
    nch = bpw // CH
    ngc = CH // L
    mesh = plsc.VectorSubcoreMesh(core_axis_name="c", subcore_axis_name="s")

    @functools.partial(
        pl.kernel,
        out_type=jax.ShapeDtypeStruct((B,), jnp.float32),
        mesh=mesh,
        scratch_types=[
            pltpu.VMEM((bpw,), jnp.int32),
            pltpu.VMEM((bpw,), jnp.int32),
            pltpu.VMEM((nch, CH), jnp.int32),
            pltpu.VMEM((nch, CH), jnp.int32),
            pltpu.VMEM((CH, BLK), jnp.float32),
            pltpu.VMEM((CH, BLK), jnp.float32),
            pltpu.VMEM((CH, BLK), jnp.float32),
            pltpu.VMEM((CH, BLK), jnp.float32),
            pltpu.VMEM((D,), jnp.float32),
            pltpu.VMEM((L,), jnp.float32),
            pltpu.VMEM((bpw,), jnp.float32),
            pltpu.SemaphoreType.DMA,
            pltpu.SemaphoreType.DMA,
        ],
    )
    def gmf(uid_hbm, iid_hbm, utab_hbm, itab_hbm, w_hbm, b_hbm, out_hbm,
            uidx, iidx, uridx, iridx, ubufa, ubufb, vbufa, vbufb,
            wv, bv, outv, sema, semb):
        wid = lax.axis_index("s") * nc + lax.axis_index("c")
        base = wid * bpw

        pltpu.sync_copy(uid_hbm.at[pl.ds(base, bpw)], uidx)
        pltpu.sync_copy(iid_hbm.at[pl.ds(base, bpw)], iidx)

        for c in range(nch):
            for g in range(ngc):
                sl = pl.ds(c * CH + g * L, L)
                dsl = pl.ds(g * L, L)
                uridx[c, dsl] = uidx[sl] >> 2
                iridx[c, dsl] = iidx[sl] >> 2

        ubufs, vbufs, sems = [ubufa, ubufb], [vbufa, vbufb], [sema, semb]

        def fire(c):
            bi = c % 2
            return [
                pltpu.async_copy(utab_hbm.at[uridx.at[c]], ubufs[bi], sems[bi]),
                pltpu.async_copy(itab_hbm.at[iridx.at[c]], vbufs[bi], sems[bi]),
            ]

        pending = {0: fire(0), 1: fire(1)}

        pltpu.sync_copy(w_hbm, wv)
        pltpu.sync_copy(b_hbm, bv)
        w_lo = wv[pl.ds(0, L)]
        w_hi = wv[pl.ds(L, L)]
        bvec = bv[...]
        lanes = lax.iota(jnp.int32, L)
        zero = jnp.zeros((L,), jnp.float32)

        for c in range(nch):
            for cp in pending.pop(c):
                cp.wait()
            ub, vb = ubufs[c % 2], vbufs[c % 2]

            def gstep(g, carry, c=c, ub=ub, vb=vb):
                sl = pl.ds(c * CH + g * L, L)
                uo = (uidx[sl] & (RPB - 1)) << 5
                io = (iidx[sl] & (RPB - 1)) << 5
                acc = zero
                for jj in range(L):
                    r = g * L + jj
                    ou = uo[jj]
                    oi = io[jj]
                    q = (ub[r, pl.ds(ou, L)] * vb[r, pl.ds(oi, L)] * w_lo
                         + ub[r, pl.ds(ou + L, L)] * vb[r, pl.ds(oi + L, L)]
                         * w_hi)
                    q = _lanesum(q)
                    acc = lax.select(lanes == jj, q + bvec, acc)
                outv[pl.ds(c * CH + g * L, L)] = acc
                return carry

            lax.fori_loop(0, ngc, gstep, 0)
            if c + 2 < nch:
                pending[c + 2] = fire(c + 2)

        pltpu.sync_copy(outv, out_hbm.at[pl.ds(base, bpw)])

    return gmf


def kernel(user_id, item_id, user_emb, item_emb, linear_w, linear_b):
    info = plsc.get_sparse_core_info()
    n_user, n_item = user_emb.shape[0], item_emb.shape[0]
    gmf = _build(info.num_cores, info.num_subcores, n_user, n_item)
    utab = jnp.reshape(user_emb, (n_user * D // BLK, BLK))
    itab = jnp.reshape(item_emb, (n_item * D // BLK, BLK))
    w = jnp.reshape(linear_w, (D,)).astype(jnp.float32)
    b = jnp.broadcast_to(jnp.reshape(linear_b, ()), (L,)).astype(jnp.float32)
    return gmf(user_id.astype(jnp.int32), item_id.astype(jnp.int32),
               utab, itab, w, b)


# speedup vs baseline: 1.5008x; 1.5008x over previous
import functools

import jax
import jax.numpy as jnp
from jax import lax
from jax.experimental import pallas as pl
from jax.experimental.pallas import tpu as pltpu
from jax.experimental.pallas import tpu_sc as plsc

B = 16384
D = 32
L = 16
NBUF = 4


def _lanesum(q):
    dn = lax.GatherDimensionNumbers(
        offset_dims=(), collapsed_slice_dims=(0,), start_index_map=(0,))
    for s in (1, 2, 4, 8):
        idx = lax.iota(jnp.int32, L) ^ s
        q = q + lax.gather(q, idx[:, None], dn, (1,),
                           mode=lax.GatherScatterMode.PROMISE_IN_BOUNDS)
    return q


def _build(nc: int, ns: int):
    nw = nc * ns
    bpw = B // nw
    ng = bpw // L
    mesh = plsc.VectorSubcoreMesh(core_axis_name="c", subcore_axis_name="s")

    @functools.partial(
        pl.kernel,
        out_type=jax.ShapeDtypeStruct((B,), jnp.float32),
        mesh=mesh,
        compiler_params=pltpu.CompilerParams(use_tc_tiling_on_sc=True),
        scratch_types=(
            [pltpu.VMEM((bpw,), jnp.int32)] * 2
            + [pltpu.VMEM((L, D), jnp.float32)] * NBUF
            + [pltpu.VMEM((L, D), jnp.float32)] * NBUF
            + [pltpu.VMEM((D,), jnp.float32),
               pltpu.VMEM((L,), jnp.float32),
               pltpu.VMEM((bpw,), jnp.float32)]
            + [pltpu.SemaphoreType.DMA] * (2 * NBUF)
        ),
    )
    def gmf(uid_hbm, iid_hbm, utab_hbm, itab_hbm, w_hbm, b_hbm, out_hbm,
            uidx, iidx, *rest):
        ubufs = rest[:NBUF]
        vbufs = rest[NBUF:2 * NBUF]
        wv, bv, outv = rest[2 * NBUF:2 * NBUF + 3]
        usems = rest[2 * NBUF + 3:2 * NBUF + 3 + NBUF]
        vsems = rest[2 * NBUF + 3 + NBUF:]

        wid = lax.axis_index("s") * nc + lax.axis_index("c")
        base = wid * bpw

        pltpu.sync_copy(uid_hbm.at[pl.ds(base, bpw)], uidx)
        pltpu.sync_copy(iid_hbm.at[pl.ds(base, bpw)], iidx)
        pltpu.sync_copy(w_hbm, wv)
        pltpu.sync_copy(b_hbm, bv)

        def fire(g, p):
            uiv = uidx[pl.ds(g * L, L)]
            iiv = iidx[pl.ds(g * L, L)]
            for j in range(L):
                pltpu.async_copy(utab_hbm.at[pl.ds(uiv[j], 1), :],
                                 ubufs[p].at[pl.ds(j, 1), :], usems[p])
                pltpu.async_copy(itab_hbm.at[pl.ds(iiv[j], 1), :],
                                 vbufs[p].at[pl.ds(j, 1), :], vsems[p])

        def drain(p):
            pltpu.make_async_copy(utab_hbm.at[pl.ds(0, L), :],
                                  ubufs[p], usems[p]).wait()
            pltpu.make_async_copy(itab_hbm.at[pl.ds(0, L), :],
                                  vbufs[p], vsems[p]).wait()

        for p in range(NBUF):
            fire(p, p)

        w_lo = wv[pl.ds(0, L)]
        w_hi = wv[pl.ds(L, L)]
        bvec = bv[...]
        lanes = lax.iota(jnp.int32, L)
        zero = jnp.zeros((L,), jnp.float32)

        def tstep(t, carry):
            for p in range(NBUF):
                g = t * NBUF + p
                drain(p)
                ub, vb = ubufs[p], vbufs[p]
                acc = zero
                for jj in range(L):
                    q = (ub[jj, pl.ds(0, L)] * vb[jj, pl.ds(0, L)] * w_lo
                         + ub[jj, pl.ds(L, L)] * vb[jj, pl.ds(L, L)] * w_hi)
                    q = _lanesum(q)
                    acc = lax.select(lanes == jj, q + bvec, acc)
                outv[pl.ds(g * L, L)] = acc

                @pl.when(g + NBUF < ng)
                def _():
                    fire(g + NBUF, p)
            return carry

        lax.fori_loop(0, ng // NBUF, tstep, 0)

        pltpu.sync_copy(outv, out_hbm.at[pl.ds(base, bpw)])

    return gmf


def kernel(user_id, item_id, user_emb, item_emb, linear_w, linear_b):
    info = plsc.get_sparse_core_info()
    gmf = _build(info.num_cores, info.num_subcores)
    w = jnp.reshape(linear_w, (D,)).astype(jnp.float32)
    b = jnp.broadcast_to(jnp.reshape(linear_b, ()), (L,)).astype(jnp.float32)
    return gmf(user_id.astype(jnp.int32), item_id.astype(jnp.int32),
               user_emb, item_emb, w, b)
